# baseline (speedup 1.0000x reference)
import functools
import jax
import jax.numpy as jnp
from jax import lax
from jax.experimental import pallas as pl
from jax.experimental.pallas import tpu as pltpu
from jax.experimental.pallas import tpu_sc as plsc

_NUM_BUCKETS = 32
_NUM_HEADS = 16
_S = 2048
_L = 4224
_THS = (12, 16, 23, 32, 46, 64, 91)
_ROWS_PER_W = _S // 2


def _sc_body(w_hbm, out_hbm, wrow_ref, v0, v1, v2, v3, v4, v5, v6, v7, sem):
    v8 = (v0, v1, v2, v3, v4, v5, v6, v7)
    cid = lax.axis_index("c")
    sid = lax.axis_index("s")
    wid = sid * 2 + cid
    h = wid // 2
    r0 = (wid % 2) * _ROWS_PER_W

    pltpu.sync_copy(w_hbm.at[h], wrow_ref)
    w_lo = wrow_ref[pl.ds(0, 16)]
    w_hi = wrow_ref[pl.ds(16, 16)]
    lane = lax.iota(jnp.int32, 16)

    def build(c, _):
        m = lane + c * 16
        d = m - (_S - 1)
        base = jnp.where(d > 0, 16, 0).astype(jnp.int32)
        rp = jnp.abs(d)
        large = jnp.full((16,), 8, jnp.int32)
        for t in _THS:
            large = large + jnp.where(rp >= t, 1, 0).astype(jnp.int32)
        bucket = base + jnp.where(rp < 8, rp, large)
        b16 = jnp.bitwise_and(bucket, 15)
        val = jnp.where(
            bucket < 16,
            w_lo.at[b16].get(mode="promise_in_bounds"),
            w_hi.at[b16].get(mode="promise_in_bounds"),
        )
        for rho in range(8):
            v8[rho][pl.ds(c * 16 + rho, 16)] = val
        return _

    lax.fori_loop(0, (_L - 16) // 16, build, 0)

    def emit(g, _):
        i_base = r0 + g * 8
        copies = []
        for k in range(8):
            i = i_base + k
            o = (_S - 1) - i
            rho = (8 - ((7 - k) & 7)) & 7
            a = o + rho
            copies.append(pltpu.make_async_copy(
                v8[rho].at[pl.ds(a, _S)],
                out_hbm.at[0, h, i, :],
                sem,
            ))
        for cp in copies:
            cp.start()
        for cp in copies:
            cp.wait()
        return _

    lax.fori_loop(0, _ROWS_PER_W // 8, emit, 0)


def kernel(input_ids, W):
    S = input_ids.shape[1]
    assert S == _S and W.shape == (_NUM_BUCKETS, _NUM_HEADS)
    mesh = plsc.VectorSubcoreMesh(core_axis_name="c", subcore_axis_name="s")
    kfun = functools.partial(
        pl.kernel,
        mesh=mesh,
        compiler_params=pltpu.CompilerParams(use_tc_tiling_on_sc=False),
        out_type=jax.ShapeDtypeStruct((1, _NUM_HEADS, _S, _S), jnp.float32),
        scratch_types=[
            pltpu.VMEM((_NUM_BUCKETS,), jnp.float32),
        ] + [pltpu.VMEM((_L,), jnp.float32) for _ in range(8)] + [
            pltpu.SemaphoreType.DMA,
        ],
    )(_sc_body)
    return kfun(W.astype(jnp.float32).T)


# speedup vs baseline: 1.0004x; 1.0004x over previous
import functools
import jax
import jax.numpy as jnp
from jax import lax
from jax.experimental import pallas as pl
from jax.experimental.pallas import tpu as pltpu
from jax.experimental.pallas import tpu_sc as plsc

_NUM_BUCKETS = 32
_NUM_HEADS = 16
_S = 2048
_L = 4224
_THS = (12, 16, 23, 32, 46, 64, 91)
_ROWS_PER_W = _S // 2


def _sc_body(w_hbm, out_hbm, wrow_ref, v0, v1, v2, v3, v4, v5, v6, v7, sem):
    v8 = (v0, v1, v2, v3, v4, v5, v6, v7)
    cid = lax.axis_index("c")
    sid = lax.axis_index("s")
    wid = sid * 2 + cid
    h = wid // 2
    r0 = (wid % 2) * _ROWS_PER_W

    pltpu.sync_copy(w_hbm.at[h], wrow_ref)
    w_lo = wrow_ref[pl.ds(0, 16)]
    w_hi = wrow_ref[pl.ds(16, 16)]
    lane = lax.iota(jnp.int32, 16)

    def build(c, _):
        m = lane + c * 16
        d = m - (_S - 1)
        base = jnp.where(d > 0, 16, 0).astype(jnp.int32)
        rp = jnp.abs(d)
        large = jnp.full((16,), 8, jnp.int32)
        for t in _THS:
            large = large + jnp.where(rp >= t, 1, 0).astype(jnp.int32)
        bucket = base + jnp.where(rp < 8, rp, large)
        b16 = jnp.bitwise_and(bucket, 15)
        val = jnp.where(
            bucket < 16,
            w_lo.at[b16].get(mode="promise_in_bounds"),
            w_hi.at[b16].get(mode="promise_in_bounds"),
        )
        for rho in range(8):
            v8[rho][pl.ds(c * 16 + rho, 16)] = val
        return _

    lax.fori_loop(0, (_L - 16) // 16, build, 0)

    def mk(g):
        copies = []
        for k in range(8):
            i = r0 + g * 8 + k
            o = (_S - 1) - i
            rho = (8 - ((7 - k) & 7)) & 7
            a = o + rho
            copies.append(pltpu.make_async_copy(
                v8[rho].at[pl.ds(a, _S)],
                out_hbm.at[0, h, i, :],
                sem,
            ))
        return copies

    ngroups = _ROWS_PER_W // 8
    for cp in mk(0) + mk(1):
        cp.start()

    def emit(g, _):
        for cp in mk(g + 2):
            cp.start()
        for cp in mk(g):
            cp.wait()
        return _

    lax.fori_loop(0, ngroups - 2, emit, 0)
    for cp in mk(ngroups - 2) + mk(ngroups - 1):
        cp.wait()


def kernel(input_ids, W):
    S = input_ids.shape[1]
    assert S == _S and W.shape == (_NUM_BUCKETS, _NUM_HEADS)
    mesh = plsc.VectorSubcoreMesh(core_axis_name="c", subcore_axis_name="s")
    kfun = functools.partial(
        pl.kernel,
        mesh=mesh,
        compiler_params=pltpu.CompilerParams(use_tc_tiling_on_sc=False),
        out_type=jax.ShapeDtypeStruct((1, _NUM_HEADS, _S, _S), jnp.float32),
        scratch_types=[
            pltpu.VMEM((_NUM_BUCKETS,), jnp.float32),
        ] + [pltpu.VMEM((_L,), jnp.float32) for _ in range(8)] + [
            pltpu.SemaphoreType.DMA,
        ],
    )(_sc_body)
    return kfun(W.astype(jnp.float32).T)


# speedup vs baseline: 1.0024x; 1.0020x over previous
import functools
import jax
import jax.numpy as jnp
from jax import lax
from jax.experimental import pallas as pl
from jax.experimental.pallas import tpu as pltpu
from jax.experimental.pallas import tpu_sc as plsc

_NUM_BUCKETS = 32
_NUM_HEADS = 16
_S = 2048
_L = 4224
_THS = (12, 16, 23, 32, 46, 64, 91)
_ROWS_PER_W = _S // 2
_NBLK = _ROWS_PER_W // 8
_RING = 4


def _sc_body(w_hbm, out_hbm, wrow_ref, s8_ref, sem):
    cid = lax.axis_index("c")
    sid = lax.axis_index("s")
    wid = sid * 2 + cid
    h = wid // 2
    r0 = (wid % 2) * _ROWS_PER_W

    pltpu.sync_copy(w_hbm.at[h], wrow_ref)
    w_lo = wrow_ref[pl.ds(0, 16)]
    w_hi = wrow_ref[pl.ds(16, 16)]
    lane = lax.iota(jnp.int32, 16)

    def build(c, _):
        m = lane + c * 16
        d = m - (_S - 1)
        base = jnp.where(d > 0, 16, 0).astype(jnp.int32)
        rp = jnp.abs(d)
        large = jnp.full((16,), 8, jnp.int32)
        for t in _THS:
            large = large + jnp.where(rp >= t, 1, 0).astype(jnp.int32)
        bucket = base + jnp.where(rp < 8, rp, large)
        b16 = jnp.bitwise_and(bucket, 15)
        val = jnp.where(
            bucket < 16,
            w_lo.at[b16].get(mode="promise_in_bounds"),
            w_hi.at[b16].get(mode="promise_in_bounds"),
        )
        for r in range(8):
            s8_ref[r, pl.ds(c * 16 + r + 1, 16)] = val
        return _

    lax.fori_loop(0, (_L - 32) // 16, build, 0)

    def mk(b):
        i0 = r0 + b * 8
        a = _S - i0
        return pltpu.make_async_copy(
            s8_ref.at[:, pl.ds(a, _S)],
            out_hbm.at[0, h, pl.ds(i0, 8), :],
            sem,
        )

    for b in range(_RING):
        mk(b).start()

    def emit(b, _):
        mk(b + _RING).start()
        mk(b).wait()
        return _

    lax.fori_loop(0, _NBLK - _RING, emit, 0)
    for b in range(_RING):
        mk(_NBLK - _RING + b).wait()


def kernel(input_ids, W):
    S = input_ids.shape[1]
    assert S == _S and W.shape == (_NUM_BUCKETS, _NUM_HEADS)
    mesh = plsc.VectorSubcoreMesh(core_axis_name="c", subcore_axis_name="s")
    kfun = functools.partial(
        pl.kernel,
        mesh=mesh,
        compiler_params=pltpu.CompilerParams(use_tc_tiling_on_sc=False),
        out_type=jax.ShapeDtypeStruct((1, _NUM_HEADS, _S, _S), jnp.float32),
        scratch_types=[
            pltpu.VMEM((_NUM_BUCKETS,), jnp.float32),
            pltpu.VMEM((8, _L), jnp.float32),
            pltpu.SemaphoreType.DMA,
        ],
    )(_sc_body)
    return kfun(W.astype(jnp.float32).T)


# speedup vs baseline: 4.3630x; 4.3527x over previous
import jax
import jax.numpy as jnp
from jax import lax
from jax.experimental import pallas as pl
from jax.experimental.pallas import tpu as pltpu

_NUM_BUCKETS = 32
_MAX_DISTANCE = 128
_NUM_HEADS = 16
_S = 2048
_TI = 256
_L = 4224


def _bucket_map(d):
    nb = _NUM_BUCKETS // 2
    ret = jnp.where(d > 0, nb, 0).astype(jnp.int32)
    rp = jnp.abs(d)
    max_exact = nb // 2
    is_small = rp < max_exact
    rel_f = rp.astype(jnp.float32)
    scale = (nb - max_exact) / jnp.log(_MAX_DISTANCE / max_exact)
    large = max_exact + (
        jnp.log(jnp.maximum(rel_f, 1.0) / max_exact) * scale
    ).astype(jnp.int32)
    large = jnp.minimum(large, nb - 1)
    return ret + jnp.where(is_small, rp, large)


def _kernel_body(w_ref, out_ref, bucket8_ref, v8r_ref):
    h = pl.program_id(0)

    @pl.when(h == 0)
    def _init_buckets():
        m = lax.broadcasted_iota(jnp.int32, (8, _L), 1)
        r = lax.broadcasted_iota(jnp.int32, (8, _L), 0)
        bucket8_ref[...] = _bucket_map(m - r - (_S - 1))

    bucket8 = bucket8_ref[...]
    acc = jnp.zeros((8, _L), jnp.float32)
    for b in range(_NUM_BUCKETS):
        acc = jnp.where(bucket8 == b, w_ref[b, h], acc)
    for k in range(16):
        rho = 8 * k + 7
        v8r_ref[k] = jnp.concatenate([acc[:, rho:], acc[:, :rho]], axis=1)

    for s in range(_S // 8):
        c0 = (_S - 1) - 8 * s
        rho = c0 % 128
        a = c0 - rho
        out_ref[0, 0, 8 * s:8 * s + 8, :] = v8r_ref[(rho - 7) // 8, :, a:a + _S]


def kernel(input_ids, W):
    S = input_ids.shape[1]
    assert S == _S and W.shape == (_NUM_BUCKETS, _NUM_HEADS)
    out = pl.pallas_call(
        _kernel_body,
        grid=(_NUM_HEADS,),
        in_specs=[pl.BlockSpec(memory_space=pltpu.SMEM)],
        out_specs=pl.BlockSpec(
            (1, 1, _S, _S), lambda h: (0, h, 0, 0)
        ),
        out_shape=jax.ShapeDtypeStruct((1, _NUM_HEADS, _S, _S), jnp.float32),
        scratch_shapes=[
            pltpu.VMEM((8, _L), jnp.int32),
            pltpu.VMEM((16, 8, _L), jnp.float32),
        ],
        compiler_params=pltpu.CompilerParams(
            dimension_semantics=("arbitrary",),
        ),
    )(W.astype(jnp.float32))
    return out
